# baseline (speedup 1.0000x reference)
import functools

import jax
import jax.numpy as jnp
from jax import lax
from jax.experimental import pallas as pl
from jax.experimental.pallas import tpu as pltpu
from jax.experimental.pallas import tpu_sc as plsc

_VOX = 96 * 96 * 96
_ROW = 96
_NROW = _VOX // _ROW
_NC, _NS, _L = 2, 16, 16
_NW = _NC * _NS
_CROW = 64
_LB = _ROW // _L

_SC_START = 7168
_SC_ROWS = _NROW - _SC_START
_SC_W = _SC_ROWS // _NS
_NCHUNK = _SC_W // _CROW

_TC_R = 896
_TC_STEPS = _SC_START // _TC_R


def _sc_partials_body(pred_hbm, tgt_hbm, out_hbm, *refs):
    bufs = refs[:14]
    obuf = refs[14]
    sems = refs[15:17]
    wid = lax.axis_index("s") * _NC + lax.axis_index("c")
    b = wid // _NS
    wbase = _SC_START + (wid % _NS) * _SC_W

    def copies(k, ring):
        base = wbase + k * _CROW
        bs = bufs[7 * ring:7 * ring + 7]
        descs = []
        for c in range(4):
            descs.append(pltpu.make_async_copy(
                pred_hbm.at[b * 4 + c, pl.ds(base, _CROW), :],
                bs[c], sems[ring]))
        for c in range(3):
            descs.append(pltpu.make_async_copy(
                tgt_hbm.at[b * 4 + 1 + c, pl.ds(base, _CROW), :],
                bs[4 + c], sems[ring]))
        return descs

    def compute(ring, accs):
        pb0, pb1, pb2, pb3, tb1, tb2, tb3 = bufs[7 * ring:7 * ring + 7]

        def body(r, accs):
            for l in range(_LB):
                (ai1, ai2, ai3, ap1, ap2, ap3, at1, at2, at3) = accs
                s = pl.ds(l * _L, _L)
                p0 = pb0[r, s]
                p1 = pb1[r, s]
                p2 = pb2[r, s]
                p3 = pb3[r, s]
                t1 = tb1[r, s]
                t2 = tb2[r, s]
                t3 = tb3[r, s]
                is1 = (p1 > p0) & (p1 >= p2) & (p1 >= p3)
                is2 = (p2 > p0) & (p2 > p1) & (p2 >= p3)
                is3 = (p3 > p0) & (p3 > p1) & (p3 > p2)
                zero = jnp.zeros((_L,), jnp.float32)
                one = jnp.ones((_L,), jnp.float32)
                accs = (
                    ai1 + jnp.where(is1, t1, zero),
                    ai2 + jnp.where(is2, t2, zero),
                    ai3 + jnp.where(is3, t3, zero),
                    ap1 + jnp.where(is1, one, zero),
                    ap2 + jnp.where(is2, one, zero),
                    ap3 + jnp.where(is3, one, zero),
                    at1 + t1,
                    at2 + t2,
                    at3 + t3,
                )
            return accs

        return plsc.parallel_loop(0, _CROW, carry=accs, unroll=2)(body)

    accs = tuple(jnp.zeros((_L,), jnp.float32) for _ in range(9))
    for d in copies(0, 0):
        d.start()
    for k in range(_NCHUNK):
        ring = k % 2
        if k + 1 < _NCHUNK:
            for d in copies(k + 1, 1 - ring):
                d.start()
        for d in copies(k, ring):
            d.wait()
        accs = compute(ring, accs)
    for k in range(9):
        obuf[k, :] = accs[k]
    pltpu.sync_copy(obuf, out_hbm.at[wid])


@functools.cache
def _sc_partials():
    return pl.kernel(
        _sc_partials_body,
        mesh=plsc.VectorSubcoreMesh(core_axis_name="c", subcore_axis_name="s"),
        out_type=jax.ShapeDtypeStruct((_NW, 9, _L), jnp.float32),
        scratch_types=[pltpu.VMEM((_CROW, _ROW), jnp.float32)] * 14
        + [pltpu.VMEM((9, _L), jnp.float32)]
        + [pltpu.SemaphoreType.DMA] * 2,
    )


def _tc_partials_body(pred_ref, tgta_ref, tgtb_ref, out_ref):
    i = pl.program_id(0)

    @pl.when(i == 0)
    def _():
        out_ref[...] = jnp.zeros_like(out_ref)

    parts = []
    for b in range(2):
        tgt_ref = (tgta_ref, tgtb_ref)[b]
        p0 = pred_ref[4 * b + 0]
        p1 = pred_ref[4 * b + 1]
        p2 = pred_ref[4 * b + 2]
        p3 = pred_ref[4 * b + 3]
        t1 = tgt_ref[0]
        t2 = tgt_ref[1]
        t3 = tgt_ref[2]
        is1 = (p1 > p0) & (p1 >= p2) & (p1 >= p3)
        is2 = (p2 > p0) & (p2 > p1) & (p2 >= p3)
        is3 = (p3 > p0) & (p3 > p1) & (p3 > p2)
        zero = jnp.zeros_like(t1)
        parts.append((
            jnp.where(is1, t1, zero),
            jnp.where(is2, t2, zero),
            jnp.where(is3, t3, zero),
            is1.astype(jnp.float32),
            is2.astype(jnp.float32),
            is3.astype(jnp.float32),
            t1, t2, t3,
        ))
    for k in range(9):
        col = jnp.sum(parts[0][k] + parts[1][k], axis=0, keepdims=True)
        out_ref[k:k + 1, :] = out_ref[k:k + 1, :] + col


@functools.cache
def _tc_partials():
    return pl.pallas_call(
        _tc_partials_body,
        grid=(_TC_STEPS,),
        in_specs=[
            pl.BlockSpec((8, _TC_R, _ROW), lambda i: (0, i, 0)),
            pl.BlockSpec((pl.Element(3), pl.Element(_TC_R), pl.Element(_ROW)),
                         lambda i: (1, i * _TC_R, 0)),
            pl.BlockSpec((pl.Element(3), pl.Element(_TC_R), pl.Element(_ROW)),
                         lambda i: (5, i * _TC_R, 0)),
        ],
        out_specs=pl.BlockSpec((16, _ROW), lambda i: (0, 0)),
        out_shape=jax.ShapeDtypeStruct((16, _ROW), jnp.float32),
    )


def _finalize_body(scp_ref, tcp_ref, pct_ref, tcl_ref, out_ref):
    scp = scp_ref[...]
    tcp = tcp_ref[...]
    s = [jnp.sum(scp[:, k, :]) + jnp.sum(tcp[k, :]) for k in range(9)]

    mdsc = jnp.float32(0.0)
    mppv = jnp.float32(0.0)
    msen = jnp.float32(0.0)
    cnt_t = jnp.float32(0.0)
    cnt_p = jnp.float32(0.0)
    for c in range(3):
        inter, psum, tsum = s[c], s[3 + c], s[6 + c]
        valid_t = (tsum > 0).astype(jnp.float32)
        valid_p = (psum > 0).astype(jnp.float32)
        cnt_t = cnt_t + valid_t
        cnt_p = cnt_p + valid_p
        mdsc = mdsc + valid_t * (2.0 * inter + 1e-5) / (psum + tsum + 1e-5)
        mppv = mppv + valid_p * (inter + 1.0) / (psum + 1.0)
        msen = msen + valid_t * (inter + 1.0) / (tsum + 1.0)
    dsc = jnp.where(cnt_t > 0, mdsc / jnp.maximum(cnt_t, 1.0), mdsc)
    ppv = jnp.where(cnt_p > 0, mppv / jnp.maximum(cnt_p, 1.0), mppv)
    sen = jnp.where(cnt_t > 0, msen / jnp.maximum(cnt_t, 1.0), msen)

    pct = pct_ref[...]
    best = pct[0:1, :]
    idx = jnp.zeros((1, 32), jnp.int32)
    for k in range(1, 4):
        row = pct[k:k + 1, :]
        take = row > best
        idx = jnp.where(take, jnp.int32(k), idx)
        best = jnp.where(take, row, best)
    eq = (idx == tcl_ref[...]).astype(jnp.float32)
    ii = lax.broadcasted_iota(jnp.int32, (1, 32), 1)
    pred_actual = jnp.sum(jnp.where(ii < 2, eq, 0.0)) / 2.0
    pred_stat = jnp.sum(jnp.where(ii >= 2, eq, 0.0)) / 30.0

    oi = lax.broadcasted_iota(jnp.int32, (1, 8), 1)
    out = (jnp.where(oi == 0, dsc, 0.0)
           + jnp.where(oi == 1, ppv, 0.0)
           + jnp.where(oi == 2, sen, 0.0)
           + jnp.where(oi == 3, pred_actual, 0.0)
           + jnp.where(oi == 4, pred_stat, 0.0))
    out_ref[...] = out


def kernel(pred_mask, pred_classes, target_mask, target_class):
    pred96 = pred_mask.reshape(8, _NROW, _ROW)
    tgt96 = target_mask.reshape(8, _NROW, _ROW)
    sc_part = _sc_partials()(pred96, tgt96)
    tc_part = _tc_partials()(pred96, tgt96, tgt96)
    pct = pred_classes.T
    tcl = target_class.astype(jnp.int32).reshape(1, 32)
    out = pl.pallas_call(
        _finalize_body,
        out_shape=jax.ShapeDtypeStruct((1, 8), jnp.float32),
    )(sc_part, tc_part, pct, tcl)
    return out[0, :5]


# speedup vs baseline: 1.0499x; 1.0499x over previous
import functools

import jax
import jax.numpy as jnp
from jax import lax
from jax.experimental import pallas as pl
from jax.experimental.pallas import tpu as pltpu
from jax.experimental.pallas import tpu_sc as plsc

_VOX = 96 * 96 * 96
_ROW = 96
_NROW = _VOX // _ROW
_NC, _NS, _L = 2, 16, 16
_NW = _NC * _NS
_CROW = 64
_LB = _ROW // _L

_SC_START = 7168
_SC_ROWS = _NROW - _SC_START
_SC_W = _SC_ROWS // _NS
_NCHUNK = _SC_W // _CROW

_TC_R = 256
_TC_STEPS = _SC_START // _TC_R


def _sc_partials_body(pred_hbm, tgt_hbm, out_hbm, *refs):
    bufs = refs[:14]
    obuf = refs[14]
    sems = refs[15:17]
    wid = lax.axis_index("s") * _NC + lax.axis_index("c")
    b = wid // _NS
    wbase = _SC_START + (wid % _NS) * _SC_W

    def copies(k, ring):
        base = wbase + k * _CROW
        bs = bufs[7 * ring:7 * ring + 7]
        descs = []
        for c in range(4):
            descs.append(pltpu.make_async_copy(
                pred_hbm.at[b * 4 + c, pl.ds(base, _CROW), :],
                bs[c], sems[ring]))
        for c in range(3):
            descs.append(pltpu.make_async_copy(
                tgt_hbm.at[b * 4 + 1 + c, pl.ds(base, _CROW), :],
                bs[4 + c], sems[ring]))
        return descs

    def compute(ring, accs):
        pb0, pb1, pb2, pb3, tb1, tb2, tb3 = bufs[7 * ring:7 * ring + 7]

        def body(r, accs):
            for l in range(_LB):
                (ai1, ai2, ai3, ap1, ap2, ap3, at1, at2, at3) = accs
                s = pl.ds(l * _L, _L)
                p0 = pb0[r, s]
                p1 = pb1[r, s]
                p2 = pb2[r, s]
                p3 = pb3[r, s]
                t1 = tb1[r, s]
                t2 = tb2[r, s]
                t3 = tb3[r, s]
                is1 = (p1 > p0) & (p1 >= p2) & (p1 >= p3)
                is2 = (p2 > p0) & (p2 > p1) & (p2 >= p3)
                is3 = (p3 > p0) & (p3 > p1) & (p3 > p2)
                zero = jnp.zeros((_L,), jnp.float32)
                one = jnp.ones((_L,), jnp.float32)
                accs = (
                    ai1 + jnp.where(is1, t1, zero),
                    ai2 + jnp.where(is2, t2, zero),
                    ai3 + jnp.where(is3, t3, zero),
                    ap1 + jnp.where(is1, one, zero),
                    ap2 + jnp.where(is2, one, zero),
                    ap3 + jnp.where(is3, one, zero),
                    at1 + t1,
                    at2 + t2,
                    at3 + t3,
                )
            return accs

        return plsc.parallel_loop(0, _CROW, carry=accs, unroll=2)(body)

    accs = tuple(jnp.zeros((_L,), jnp.float32) for _ in range(9))
    for d in copies(0, 0):
        d.start()
    for k in range(_NCHUNK):
        ring = k % 2
        if k + 1 < _NCHUNK:
            for d in copies(k + 1, 1 - ring):
                d.start()
        for d in copies(k, ring):
            d.wait()
        accs = compute(ring, accs)
    for k in range(9):
        obuf[k, :] = accs[k]
    pltpu.sync_copy(obuf, out_hbm.at[wid])


@functools.cache
def _sc_partials():
    return pl.kernel(
        _sc_partials_body,
        mesh=plsc.VectorSubcoreMesh(core_axis_name="c", subcore_axis_name="s"),
        out_type=jax.ShapeDtypeStruct((_NW, 9, _L), jnp.float32),
        scratch_types=[pltpu.VMEM((_CROW, _ROW), jnp.float32)] * 14
        + [pltpu.VMEM((9, _L), jnp.float32)]
        + [pltpu.SemaphoreType.DMA] * 2,
    )


def _tc_partials_body(pred_ref, tgta_ref, tgtb_ref, out_ref):
    i = pl.program_id(0)

    @pl.when(i == 0)
    def _():
        out_ref[...] = jnp.zeros_like(out_ref)

    parts = []
    for b in range(2):
        tgt_ref = (tgta_ref, tgtb_ref)[b]
        p0 = pred_ref[4 * b + 0]
        p1 = pred_ref[4 * b + 1]
        p2 = pred_ref[4 * b + 2]
        p3 = pred_ref[4 * b + 3]
        t1 = tgt_ref[0]
        t2 = tgt_ref[1]
        t3 = tgt_ref[2]
        is1 = (p1 > p0) & (p1 >= p2) & (p1 >= p3)
        is2 = (p2 > p0) & (p2 > p1) & (p2 >= p3)
        is3 = (p3 > p0) & (p3 > p1) & (p3 > p2)
        zero = jnp.zeros_like(t1)
        parts.append((
            jnp.where(is1, t1, zero),
            jnp.where(is2, t2, zero),
            jnp.where(is3, t3, zero),
            is1.astype(jnp.float32),
            is2.astype(jnp.float32),
            is3.astype(jnp.float32),
            t1, t2, t3,
        ))
    for k in range(9):
        col = jnp.sum(parts[0][k] + parts[1][k], axis=0, keepdims=True)
        out_ref[k:k + 1, :] = out_ref[k:k + 1, :] + col


@functools.cache
def _tc_partials():
    return pl.pallas_call(
        _tc_partials_body,
        grid=(_TC_STEPS,),
        in_specs=[
            pl.BlockSpec((8, _TC_R, _ROW), lambda i: (0, i, 0)),
            pl.BlockSpec((pl.Element(3), pl.Element(_TC_R), pl.Element(_ROW)),
                         lambda i: (1, i * _TC_R, 0)),
            pl.BlockSpec((pl.Element(3), pl.Element(_TC_R), pl.Element(_ROW)),
                         lambda i: (5, i * _TC_R, 0)),
        ],
        out_specs=pl.BlockSpec((16, _ROW), lambda i: (0, 0)),
        out_shape=jax.ShapeDtypeStruct((16, _ROW), jnp.float32),
    )


def _finalize_body(scp_ref, tcp_ref, pct_ref, tcl_ref, out_ref):
    scp = scp_ref[...]
    tcp = tcp_ref[...]
    s = [jnp.sum(scp[:, k, :]) + jnp.sum(tcp[k, :]) for k in range(9)]

    mdsc = jnp.float32(0.0)
    mppv = jnp.float32(0.0)
    msen = jnp.float32(0.0)
    cnt_t = jnp.float32(0.0)
    cnt_p = jnp.float32(0.0)
    for c in range(3):
        inter, psum, tsum = s[c], s[3 + c], s[6 + c]
        valid_t = (tsum > 0).astype(jnp.float32)
        valid_p = (psum > 0).astype(jnp.float32)
        cnt_t = cnt_t + valid_t
        cnt_p = cnt_p + valid_p
        mdsc = mdsc + valid_t * (2.0 * inter + 1e-5) / (psum + tsum + 1e-5)
        mppv = mppv + valid_p * (inter + 1.0) / (psum + 1.0)
        msen = msen + valid_t * (inter + 1.0) / (tsum + 1.0)
    dsc = jnp.where(cnt_t > 0, mdsc / jnp.maximum(cnt_t, 1.0), mdsc)
    ppv = jnp.where(cnt_p > 0, mppv / jnp.maximum(cnt_p, 1.0), mppv)
    sen = jnp.where(cnt_t > 0, msen / jnp.maximum(cnt_t, 1.0), msen)

    pct = pct_ref[...]
    best = pct[0:1, :]
    idx = jnp.zeros((1, 32), jnp.int32)
    for k in range(1, 4):
        row = pct[k:k + 1, :]
        take = row > best
        idx = jnp.where(take, jnp.int32(k), idx)
        best = jnp.where(take, row, best)
    eq = (idx == tcl_ref[...]).astype(jnp.float32)
    ii = lax.broadcasted_iota(jnp.int32, (1, 32), 1)
    pred_actual = jnp.sum(jnp.where(ii < 2, eq, 0.0)) / 2.0
    pred_stat = jnp.sum(jnp.where(ii >= 2, eq, 0.0)) / 30.0

    oi = lax.broadcasted_iota(jnp.int32, (1, 8), 1)
    out = (jnp.where(oi == 0, dsc, 0.0)
           + jnp.where(oi == 1, ppv, 0.0)
           + jnp.where(oi == 2, sen, 0.0)
           + jnp.where(oi == 3, pred_actual, 0.0)
           + jnp.where(oi == 4, pred_stat, 0.0))
    out_ref[...] = out


def kernel(pred_mask, pred_classes, target_mask, target_class):
    pred96 = pred_mask.reshape(8, _NROW, _ROW)
    tgt96 = target_mask.reshape(8, _NROW, _ROW)
    sc_part = _sc_partials()(pred96, tgt96)
    tc_part = _tc_partials()(pred96, tgt96, tgt96)
    pct = pred_classes.T
    tcl = target_class.astype(jnp.int32).reshape(1, 32)
    out = pl.pallas_call(
        _finalize_body,
        out_shape=jax.ShapeDtypeStruct((1, 8), jnp.float32),
    )(sc_part, tc_part, pct, tcl)
    return out[0, :5]
